# baseline (speedup 1.0000x reference)
import functools

import jax
import jax.numpy as jnp
from jax import lax
from jax.experimental import pallas as pl
from jax.experimental.pallas import tpu as pltpu
from jax.experimental.pallas import tpu_sc as plsc

N = 10000
E = 160000
D = 256
DH = 128
NC = 2
NS = 16
NPAD = 10112
ROWS_PER_TILE = NPAD // NS
EDGES_PER_TILE = E // NS
CHUNK = 64
EPAD = 163840
EDGES_PER_TILE_PAD = EPAD // NS
K = EDGES_PER_TILE_PAD // CHUNK
G = 5
KG = K // G
NBUF = 4
TRASH = 10100
ZROWS = 32


def _sc_agg_body(xlo, xhi, e4, alo, ahi,
                 src_all, dst_all, rows_a, rows_b, rows_c, rows_d, zbuf, acc,
                 sem_a, sem_b, sem_c, sem_d):
    c = lax.axis_index("c")
    s = lax.axis_index("s")

    rows = (rows_a, rows_b, rows_c, rows_d)
    sems = (sem_a, sem_b, sem_c, sem_d)

    def start_gather(j, buf, sm):
        @pl.when(c == 0)
        def _g0():
            pltpu.async_copy(xlo.at[src_all.at[j]], buf, sm)

        @pl.when(c == 1)
        def _g1():
            pltpu.async_copy(xhi.at[src_all.at[j]], buf, sm)

    def wait_gather(buf, sm):
        pltpu.make_async_copy(xlo.at[src_all.at[0]], buf, sm).wait()

    pltpu.sync_copy(e4.at[0, s, 0], src_all)
    pltpu.sync_copy(e4.at[1, s, 0], dst_all)
    for b in range(NBUF):
        start_gather(b, rows[b], sems[b])

    zv = jnp.zeros((16,), jnp.float32)

    def zrow(i, carry):
        for j in range(DH // 16):
            zbuf[i, pl.ds(j * 16, 16)] = zv
        return carry

    lax.fori_loop(0, ZROWS, zrow, 0)
    for k2 in range(ROWS_PER_TILE // ZROWS):
        pltpu.sync_copy(zbuf, acc.at[pl.ds(s * ROWS_PER_TILE + k2 * ZROWS, ZROWS)])
    _tail = ROWS_PER_TILE % ZROWS
    if _tail:
        pltpu.sync_copy(
            zbuf.at[pl.ds(0, _tail)],
            acc.at[pl.ds(s * ROWS_PER_TILE + (ROWS_PER_TILE // ZROWS) * ZROWS,
                         _tail)])

    plsc.subcore_barrier()

    def consume(j, b):
        wait_gather(rows[b], sems[b])
        pltpu.sync_copy(rows[b], acc.at[dst_all.at[j]], add=True)

        @pl.when(j + NBUF < KG)
        def _nxt():
            start_gather(j + NBUF, rows[b], sems[b])

    for h in range(G):
        if h > 0:
            pltpu.sync_copy(e4.at[0, s, h], src_all)
            pltpu.sync_copy(e4.at[1, s, h], dst_all)
            for b in range(NBUF):
                start_gather(b, rows[b], sems[b])

        def group(i, carry):
            for b in range(NBUF):
                consume(NBUF * i + b, b)
            return carry

        lax.fori_loop(0, KG // NBUF, group, 0)
        for j in range((KG // NBUF) * NBUF, KG):
            consume(j, j % NBUF)

    plsc.subcore_barrier()

    off = s * ROWS_PER_TILE

    @pl.when(c == 0)
    def _w0():
        pltpu.sync_copy(acc.at[pl.ds(off, ROWS_PER_TILE)],
                        alo.at[pl.ds(off, ROWS_PER_TILE)])

    @pl.when(c == 1)
    def _w1():
        pltpu.sync_copy(acc.at[pl.ds(off, ROWS_PER_TILE)],
                        ahi.at[pl.ds(off, ROWS_PER_TILE)])


_sc_agg = pl.kernel(
    _sc_agg_body,
    out_type=[jax.ShapeDtypeStruct((NPAD, DH), jnp.float32),
              jax.ShapeDtypeStruct((NPAD, DH), jnp.float32)],
    mesh=plsc.VectorSubcoreMesh(core_axis_name="c", subcore_axis_name="s"),
    scratch_types=[
        pltpu.VMEM((KG, CHUNK), jnp.int32),
        pltpu.VMEM((KG, CHUNK), jnp.int32),
        pltpu.VMEM((CHUNK, DH), jnp.float32),
        pltpu.VMEM((CHUNK, DH), jnp.float32),
        pltpu.VMEM((CHUNK, DH), jnp.float32),
        pltpu.VMEM((CHUNK, DH), jnp.float32),
        pltpu.VMEM((ZROWS, DH), jnp.float32),
        pltpu.VMEM_SHARED((NPAD, DH), jnp.float32),
        pltpu.SemaphoreType.DMA,
        pltpu.SemaphoreType.DMA,
        pltpu.SemaphoreType.DMA,
        pltpu.SemaphoreType.DMA,
    ],
)


BLK = 1000
_HI = jax.lax.Precision.HIGHEST


def _tc_pre_body(x_ref, wupd_ref, wgx_ref, b_ref, t1_ref, t2_ref):
    x = x_ref[...]
    t1 = jnp.dot(x, wupd_ref[...]) + b_ref[0:1, :]
    t2 = jnp.dot(x, wgx_ref[...]) + b_ref[1:2, :]
    t1_ref[...] = t1.astype(jnp.bfloat16)
    t2_ref[...] = t2.astype(jnp.bfloat16)


_tc_pre = pl.pallas_call(
    _tc_pre_body,
    grid=(N // BLK,),
    in_specs=[
        pl.BlockSpec((BLK, D), lambda i: (i, 0)),
        pl.BlockSpec((D, D), lambda i: (0, 0)),
        pl.BlockSpec((D, D), lambda i: (1, 0)),
        pl.BlockSpec((2, D), lambda i: (0, 0)),
    ],
    out_specs=[pl.BlockSpec((BLK, D), lambda i: (i, 0)),
               pl.BlockSpec((BLK, D), lambda i: (i, 0))],
    out_shape=[jax.ShapeDtypeStruct((N, D), jnp.bfloat16),
               jax.ShapeDtypeStruct((N, D), jnp.bfloat16)],
)


def _tc_post_body(x_ref, t1_ref, t2_ref, alo_ref, ahi_ref, wgnn_ref, wgu_ref,
                  out_ref):
    x = x_ref[...]
    z = (jnp.dot(alo_ref[...], wgnn_ref[0:DH, :])
         + jnp.dot(ahi_ref[...], wgnn_ref[DH:D, :]))
    u = t1_ref[...].astype(jnp.float32) + z
    g = jax.nn.sigmoid(jnp.dot(u, wgu_ref[...]) + t2_ref[...].astype(jnp.float32))
    out_ref[...] = jnp.tanh(u) * g + x * (1.0 - g)


_tc_post = pl.pallas_call(
    _tc_post_body,
    grid=(N // BLK,),
    in_specs=[
        pl.BlockSpec((BLK, D), lambda i: (i, 0)),
        pl.BlockSpec((BLK, D), lambda i: (i, 0)),
        pl.BlockSpec((BLK, D), lambda i: (i, 0)),
        pl.BlockSpec((BLK, DH), lambda i: (i, 0)),
        pl.BlockSpec((BLK, DH), lambda i: (i, 0)),
        pl.BlockSpec((D, D), lambda i: (0, 0)),
        pl.BlockSpec((D, D), lambda i: (0, 0)),
    ],
    out_specs=pl.BlockSpec((BLK, D), lambda i: (i, 0)),
    out_shape=jax.ShapeDtypeStruct((N, D), jnp.float32),
)


def kernel(x, W_gnn, b_gnn, W_upd, b_upd, W_gate, b_gate, edge_index):
    x_lo = x[:, :DH]
    x_hi = x[:, DH:]
    epad = jnp.concatenate(
        [edge_index,
         jnp.concatenate([jnp.zeros((1, EPAD - E), jnp.int32),
                          jnp.full((1, EPAD - E), TRASH, jnp.int32)], axis=0)],
        axis=1)
    e4 = epad.reshape(2, NS, G, KG, CHUNK)
    agg_lo, agg_hi = _sc_agg(x_lo, x_hi, e4)
    b = jnp.stack([b_gnn + b_upd, b_gate], axis=0)
    t1, t2 = _tc_pre(x, W_upd, W_gate, b)
    return _tc_post(x, t1, t2, agg_lo, agg_hi, W_gnn, W_gate)


# speedup vs baseline: 2.3292x; 2.3292x over previous
import jax
import jax.numpy as jnp
from jax import lax
from jax.experimental import pallas as pl
from jax.experimental.pallas import tpu as pltpu
from jax.experimental.pallas import tpu_sc as plsc

N = 10000
E = 160000
D = 256
DH = 128
NC = 2
NS = 16
NPAD = 10112
ROWS_PER_TILE = NPAD // NS
EDGES_PER_TILE = E // NS
CHUNK = 80
K = EDGES_PER_TILE // CHUNK
G = 5
KG = K // G
NBUF = 3
ZROWS = 32


def _sc_agg_body(xlo, xhi, e4, alo, ahi,
                 src_all, dst_all, rows_a, rows_b, rows_c, zbuf, acc,
                 sem_a, sem_b, sem_c):
    c = lax.axis_index("c")
    s = lax.axis_index("s")

    rows = (rows_a, rows_b, rows_c)
    sems = (sem_a, sem_b, sem_c)

    def start_gather(j, buf, sm):
        @pl.when(c == 0)
        def _g0():
            pltpu.async_copy(xlo.at[src_all.at[j]], buf, sm)

        @pl.when(c == 1)
        def _g1():
            pltpu.async_copy(xhi.at[src_all.at[j]], buf, sm)

    def wait_gather(buf, sm):
        pltpu.make_async_copy(xlo.at[src_all.at[0]], buf, sm).wait()

    pltpu.sync_copy(e4.at[0, s, 0], src_all)
    pltpu.sync_copy(e4.at[1, s, 0], dst_all)
    for b in range(NBUF):
        start_gather(b, rows[b], sems[b])

    zv = jnp.zeros((16,), jnp.float32)

    def zrow(i, carry):
        for j in range(DH // 16):
            zbuf[i, pl.ds(j * 16, 16)] = zv
        return carry

    lax.fori_loop(0, ZROWS, zrow, 0)
    for k2 in range(ROWS_PER_TILE // ZROWS):
        pltpu.sync_copy(zbuf, acc.at[pl.ds(s * ROWS_PER_TILE + k2 * ZROWS, ZROWS)])
    _tail = ROWS_PER_TILE % ZROWS
    if _tail:
        pltpu.sync_copy(
            zbuf.at[pl.ds(0, _tail)],
            acc.at[pl.ds(s * ROWS_PER_TILE + (ROWS_PER_TILE // ZROWS) * ZROWS,
                         _tail)])

    plsc.subcore_barrier()

    def consume(j, b):
        wait_gather(rows[b], sems[b])
        pltpu.sync_copy(rows[b], acc.at[dst_all.at[j]], add=True)

        @pl.when(j + NBUF < KG)
        def _nxt():
            start_gather(j + NBUF, rows[b], sems[b])

    for h in range(G):
        if h > 0:
            pltpu.sync_copy(e4.at[0, s, h], src_all)
            pltpu.sync_copy(e4.at[1, s, h], dst_all)
            for b in range(NBUF):
                start_gather(b, rows[b], sems[b])

        def group(i, carry):
            for b in range(NBUF):
                consume(NBUF * i + b, b)
            return carry

        lax.fori_loop(0, KG // NBUF, group, 0)
        for j in range((KG // NBUF) * NBUF, KG):
            consume(j, j % NBUF)

    plsc.subcore_barrier()

    off = s * ROWS_PER_TILE

    @pl.when(c == 0)
    def _w0():
        pltpu.sync_copy(acc.at[pl.ds(off, ROWS_PER_TILE)],
                        alo.at[pl.ds(off, ROWS_PER_TILE)])

    @pl.when(c == 1)
    def _w1():
        pltpu.sync_copy(acc.at[pl.ds(off, ROWS_PER_TILE)],
                        ahi.at[pl.ds(off, ROWS_PER_TILE)])


_sc_agg = pl.kernel(
    _sc_agg_body,
    out_type=[jax.ShapeDtypeStruct((NPAD, DH), jnp.float32),
              jax.ShapeDtypeStruct((NPAD, DH), jnp.float32)],
    mesh=plsc.VectorSubcoreMesh(core_axis_name="c", subcore_axis_name="s"),
    scratch_types=[
        pltpu.VMEM((KG, CHUNK), jnp.int32),
        pltpu.VMEM((KG, CHUNK), jnp.int32),
        pltpu.VMEM((CHUNK, DH), jnp.float32),
        pltpu.VMEM((CHUNK, DH), jnp.float32),
        pltpu.VMEM((CHUNK, DH), jnp.float32),
        pltpu.VMEM((ZROWS, DH), jnp.float32),
        pltpu.VMEM_SHARED((NPAD, DH), jnp.float32),
        pltpu.SemaphoreType.DMA,
        pltpu.SemaphoreType.DMA,
        pltpu.SemaphoreType.DMA,
    ],
)


BLK = 1000


def _tc_pre_body(x_ref, wupd_ref, wgx_ref, b_ref, t1_ref, t2_ref):
    x = x_ref[...]
    t1 = jnp.dot(x, wupd_ref[...]) + b_ref[0:1, :]
    t2 = jnp.dot(x, wgx_ref[...]) + b_ref[1:2, :]
    t1_ref[...] = t1.astype(jnp.bfloat16)
    t2_ref[...] = t2.astype(jnp.bfloat16)


_tc_pre = pl.pallas_call(
    _tc_pre_body,
    grid=(N // BLK,),
    in_specs=[
        pl.BlockSpec((BLK, D), lambda i: (i, 0)),
        pl.BlockSpec((D, D), lambda i: (0, 0)),
        pl.BlockSpec((D, D), lambda i: (1, 0)),
        pl.BlockSpec((2, D), lambda i: (0, 0)),
    ],
    out_specs=[pl.BlockSpec((BLK, D), lambda i: (i, 0)),
               pl.BlockSpec((BLK, D), lambda i: (i, 0))],
    out_shape=[jax.ShapeDtypeStruct((N, D), jnp.bfloat16),
               jax.ShapeDtypeStruct((N, D), jnp.bfloat16)],
)


def _tc_post_body(x_ref, t1_ref, t2_ref, alo_ref, ahi_ref, wgnn_ref, wgu_ref,
                  out_ref):
    x = x_ref[...]
    z = (jnp.dot(alo_ref[...], wgnn_ref[0:DH, :])
         + jnp.dot(ahi_ref[...], wgnn_ref[DH:D, :]))
    u = t1_ref[...].astype(jnp.float32) + z
    g = jax.nn.sigmoid(jnp.dot(u, wgu_ref[...]) + t2_ref[...].astype(jnp.float32))
    out_ref[...] = jnp.tanh(u) * g + x * (1.0 - g)


_tc_post = pl.pallas_call(
    _tc_post_body,
    grid=(N // BLK,),
    in_specs=[
        pl.BlockSpec((BLK, D), lambda i: (i, 0)),
        pl.BlockSpec((BLK, D), lambda i: (i, 0)),
        pl.BlockSpec((BLK, D), lambda i: (i, 0)),
        pl.BlockSpec((BLK, DH), lambda i: (i, 0)),
        pl.BlockSpec((BLK, DH), lambda i: (i, 0)),
        pl.BlockSpec((D, D), lambda i: (0, 0)),
        pl.BlockSpec((D, D), lambda i: (0, 0)),
    ],
    out_specs=pl.BlockSpec((BLK, D), lambda i: (i, 0)),
    out_shape=jax.ShapeDtypeStruct((N, D), jnp.float32),
)


def kernel(x, W_gnn, b_gnn, W_upd, b_upd, W_gate, b_gate, edge_index):
    x_lo = x[:, :DH]
    x_hi = x[:, DH:]
    e4 = edge_index.reshape(2, NS, G, KG, CHUNK)
    agg_lo, agg_hi = _sc_agg(x_lo, x_hi, e4)
    b = jnp.stack([b_gnn + b_upd, b_gate], axis=0)
    t1, t2 = _tc_pre(x, W_upd, W_gate, b)
    return _tc_post(x, t1, t2, agg_lo, agg_hi, W_gnn, W_gate)
